# baseline (speedup 1.0000x reference)
import jax
import jax.numpy as jnp
from jax.experimental import pallas as pl
from jax.experimental.pallas import tpu as pltpu

D = 2048
E = 8
NB = 2
H = D // 3
TOPK = 2
N = 128
OUT = 1000

_F32 = jnp.float32
_INV_SQRT2 = 0.7071067811865476


def _gelu(x):
    return 0.5 * x * (1.0 + jax.lax.erf(x * _INV_SQRT2))


def _gate_kernel(x_ref, w1a_ref, w1b_ref, w1c_ref, w1d_ref, b1_ref, w2_ref,
                 b2_ref, w3_ref, b3_ref, ebias_ref, aw_ref, ti_ref,
                 wdense_ref):
    x = x_ref[...]
    q = D // 4
    w1s = (w1a_ref, w1b_ref, w1c_ref, w1d_ref)
    h = sum(jnp.dot(x[:, i * q:(i + 1) * q], w1s[i][...],
                    preferred_element_type=_F32)
            for i in range(4)) + b1_ref[...]
    h = jnp.maximum(h, 0.0)
    h = jnp.dot(h, w2_ref[...], preferred_element_type=_F32) + b2_ref[...]
    h = jnp.maximum(h, 0.0)
    logits = (jnp.dot(h, w3_ref[...], preferred_element_type=_F32)
              + b3_ref[...] + ebias_ref[...])
    logits = logits - jnp.max(logits, axis=-1, keepdims=True)
    ex = jnp.exp(logits)
    aw = ex / jnp.sum(ex, axis=-1, keepdims=True)
    aw_ref[...] = aw

    lane = jax.lax.broadcasted_iota(jnp.int32, aw.shape, 1)
    m1 = jnp.max(aw, axis=-1, keepdims=True)
    i1 = jnp.min(jnp.where(aw == m1, lane, E), axis=-1, keepdims=True)
    masked = jnp.where(lane == i1, -jnp.inf, aw)
    m2 = jnp.max(masked, axis=-1, keepdims=True)
    i2 = jnp.min(jnp.where(masked == m2, lane, E), axis=-1, keepdims=True)
    s = m1 + m2
    col = jax.lax.broadcasted_iota(jnp.int32, (N, TOPK), 1)
    ti_ref[...] = jnp.where(col == 0, i1, i2)
    wdense_ref[...] = (jnp.where(lane == i1, m1 / s, 0.0)
                       + jnp.where(lane == i2, m2 / s, 0.0))


def _expert_kernel(x_ref, ls_ref, lb_ref, w1a_ref, w1b_ref, w1c_ref, w1d_ref,
                   w1e_ref, w1f_ref, w1g_ref, w1h_ref, b1_ref, w2a_ref,
                   w2b_ref, w2c_ref, w2d_ref, b2_ref, wdense_ref, out_ref,
                   xe_ref):
    e = pl.program_id(0)
    nb = pl.program_id(1)

    @pl.when(jnp.logical_and(e == 0, nb == 0))
    def _init():
        out_ref[...] = jnp.zeros_like(out_ref)

    @pl.when(nb == 0)
    def _start():
        xe_ref[...] = x_ref[...]

    xe = xe_ref[...]
    mu = jnp.mean(xe, axis=-1, keepdims=True)
    cen = xe - mu
    var = jnp.mean(cen * cen, axis=-1, keepdims=True)
    hh = cen * jax.lax.rsqrt(var + 1e-5) * ls_ref[0, 0] + lb_ref[0, 0]
    q = D // 8
    w1s = (w1a_ref, w1b_ref, w1c_ref, w1d_ref, w1e_ref, w1f_ref, w1g_ref,
           w1h_ref)
    hh = sum(jnp.dot(hh[:, i * q:(i + 1) * q], w1s[i][0, 0],
                     preferred_element_type=_F32)
             for i in range(8)) + b1_ref[0, 0]
    hh = _gelu(hh)
    w2s = (w2a_ref, w2b_ref, w2c_ref, w2d_ref)
    hh = jnp.concatenate(
        [jnp.dot(hh, w2s[i][0, 0], preferred_element_type=_F32)
         for i in range(4)], axis=-1) + b2_ref[0, 0]
    xe = xe + hh
    xe_ref[...] = xe

    @pl.when(nb == NB - 1)
    def _acc():
        w = wdense_ref[...]
        lane = jax.lax.broadcasted_iota(jnp.int32, w.shape, 1)
        wcol = jnp.sum(jnp.where(lane == e, w, 0.0), axis=-1, keepdims=True)
        out_ref[...] += wcol * xe


def _head_kernel(c_ref, w1a_ref, w1b_ref, w1c_ref, w1d_ref, w1e_ref, w1f_ref,
                 w1g_ref, w1h_ref, b1_ref, w2a_ref, w2b_ref, w2c_ref,
                 w2d_ref, b2_ref, out_ref):
    c = c_ref[...]
    q = D // 8
    w1s = (w1a_ref, w1b_ref, w1c_ref, w1d_ref, w1e_ref, w1f_ref, w1g_ref,
           w1h_ref)
    ph = sum(jnp.dot(c[:, i * q:(i + 1) * q], w1s[i][...],
                     preferred_element_type=_F32)
             for i in range(8)) + b1_ref[...]
    ph = _gelu(ph)
    q2 = D // 4
    w2s = (w2a_ref, w2b_ref, w2c_ref, w2d_ref)
    out_ref[...] = sum(
        jnp.dot(ph[:, i * q2:(i + 1) * q2], w2s[i][...],
                preferred_element_type=_F32)
        for i in range(4)) + b2_ref[...]


@jax.jit
def kernel(ecfp_count_fp, gate_W1, gate_b1, gate_W2, gate_b2, gate_W3,
           gate_b3, expert_bias, ln_scale, ln_bias, eW1, eb1, eW2, eb2,
           pW1, pb1, pW2, pb2):
    x = ecfp_count_fp

    g_slab = lambda i: pl.BlockSpec((D // 4, 512), lambda g: (i, 0))
    g_full = lambda shape: pl.BlockSpec(
        shape, lambda g: (0,) * len(shape))
    all_weights, top_i, wdense = pl.pallas_call(
        _gate_kernel,
        grid=(1,),
        in_specs=[
            g_full((N, D)),
            g_slab(0), g_slab(1), g_slab(2), g_slab(3),
            g_full((1, 512)),
            g_full((512, 128)),
            g_full((1, 128)),
            g_full((128, E)),
            g_full((1, E)),
            g_full((1, E)),
        ],
        out_specs=(g_full((N, E)), g_full((N, TOPK)), g_full((N, E))),
        out_shape=(
            jax.ShapeDtypeStruct((N, E), _F32),
            jax.ShapeDtypeStruct((N, TOPK), jnp.int32),
            jax.ShapeDtypeStruct((N, E), _F32),
        ),
    )(x, gate_W1, gate_W1, gate_W1, gate_W1, gate_b1.reshape(1, -1),
      gate_W2, gate_b2.reshape(1, -1), gate_W3, gate_b3.reshape(1, -1),
      expert_bias.reshape(1, -1))

    full = lambda shape: pl.BlockSpec(shape, lambda e, nb: (0,) * len(shape))
    per_eb = lambda shape: pl.BlockSpec(
        (1, 1) + shape, lambda e, nb: (e, nb) + (0,) * len(shape))
    w1_slab = lambda i: pl.BlockSpec(
        (1, 1, D // 8, H), lambda e, nb: (e, nb, i, 0))
    w2_slab = lambda i: pl.BlockSpec(
        (1, 1, H, D // 4), lambda e, nb: (e, nb, 0, i))

    combined = pl.pallas_call(
        _expert_kernel,
        grid=(E, NB),
        in_specs=[
            full((N, D)),
            per_eb((1, D)),
            per_eb((1, D)),
            w1_slab(0), w1_slab(1), w1_slab(2), w1_slab(3),
            w1_slab(4), w1_slab(5), w1_slab(6), w1_slab(7),
            per_eb((1, H)),
            w2_slab(0), w2_slab(1), w2_slab(2), w2_slab(3),
            per_eb((1, D)),
            full((N, E)),
        ],
        out_specs=full((N, D)),
        out_shape=jax.ShapeDtypeStruct((N, D), _F32),
        scratch_shapes=[pltpu.VMEM((N, D), _F32)],
    )(x, ln_scale.reshape(E, NB, 1, D), ln_bias.reshape(E, NB, 1, D),
      eW1, eW1, eW1, eW1, eW1, eW1, eW1, eW1, eb1.reshape(E, NB, 1, H),
      eW2, eW2, eW2, eW2, eb2.reshape(E, NB, 1, D), wdense)

    p1_slab = lambda i: pl.BlockSpec((D // 8, D), lambda g: (i, 0))
    p2_slab = lambda i: pl.BlockSpec((D // 4, OUT), lambda g: (i, 0))
    spectrum = pl.pallas_call(
        _head_kernel,
        grid=(1,),
        in_specs=[
            g_full((N, D)),
            p1_slab(0), p1_slab(1), p1_slab(2), p1_slab(3),
            p1_slab(4), p1_slab(5), p1_slab(6), p1_slab(7),
            g_full((1, D)),
            p2_slab(0), p2_slab(1), p2_slab(2), p2_slab(3),
            g_full((1, OUT)),
        ],
        out_specs=g_full((N, OUT)),
        out_shape=jax.ShapeDtypeStruct((N, OUT), _F32),
    )(combined, pW1, pW1, pW1, pW1, pW1, pW1, pW1, pW1,
      pb1.reshape(1, -1), pW2, pW2, pW2, pW2, pb2.reshape(1, -1))

    return (spectrum, all_weights, top_i)


# speedup vs baseline: 1.0095x; 1.0095x over previous
import jax
import jax.numpy as jnp
from jax.experimental import pallas as pl
from jax.experimental.pallas import tpu as pltpu

D = 2048
E = 8
NB = 2
H = D // 3
TOPK = 2
N = 128
OUT = 1000

_F32 = jnp.float32
_INV_SQRT2 = 0.7071067811865476


def _gelu(x):
    return 0.5 * x * (1.0 + jax.lax.erf(x * _INV_SQRT2))


def _main_kernel(x_ref, gw1a_ref, gw1b_ref, gw1c_ref, gw1d_ref, gb1_ref,
                 gw2_ref, gb2_ref, gw3_ref, gb3_ref, ebias_ref, ls_ref,
                 lb_ref, w1a_ref, w1b_ref, w1c_ref, w1d_ref, w1e_ref,
                 w1f_ref, w1g_ref, w1h_ref, b1_ref, w2a_ref, w2b_ref,
                 w2c_ref, w2d_ref, b2_ref, out_ref, aw_ref, ti_ref,
                 xe_ref, wdense_ref):
    e = pl.program_id(0)
    nb = pl.program_id(1)

    @pl.when(jnp.logical_and(e == 0, nb == 0))
    def _gate():
        out_ref[...] = jnp.zeros_like(out_ref)
        x = x_ref[...]
        q = D // 4
        gw1s = (gw1a_ref, gw1b_ref, gw1c_ref, gw1d_ref)
        h = sum(jnp.dot(x[:, i * q:(i + 1) * q], gw1s[i][...],
                        preferred_element_type=_F32)
                for i in range(4)) + gb1_ref[...]
        h = jnp.maximum(h, 0.0)
        h = (jnp.dot(h, gw2_ref[...], preferred_element_type=_F32)
             + gb2_ref[...])
        h = jnp.maximum(h, 0.0)
        logits = (jnp.dot(h, gw3_ref[...], preferred_element_type=_F32)
                  + gb3_ref[...] + ebias_ref[...])
        logits = logits - jnp.max(logits, axis=-1, keepdims=True)
        ex = jnp.exp(logits)
        aw = ex / jnp.sum(ex, axis=-1, keepdims=True)
        aw_ref[...] = aw

        lane = jax.lax.broadcasted_iota(jnp.int32, aw.shape, 1)
        m1 = jnp.max(aw, axis=-1, keepdims=True)
        i1 = jnp.min(jnp.where(aw == m1, lane, E), axis=-1, keepdims=True)
        masked = jnp.where(lane == i1, -jnp.inf, aw)
        m2 = jnp.max(masked, axis=-1, keepdims=True)
        i2 = jnp.min(jnp.where(masked == m2, lane, E), axis=-1,
                     keepdims=True)
        s = m1 + m2
        col = jax.lax.broadcasted_iota(jnp.int32, (N, TOPK), 1)
        ti_ref[...] = jnp.where(col == 0, i1, i2)
        wdense_ref[...] = (jnp.where(lane == i1, m1 / s, 0.0)
                           + jnp.where(lane == i2, m2 / s, 0.0))

    @pl.when(nb == 0)
    def _start():
        xe_ref[...] = x_ref[...]

    xe = xe_ref[...]
    mu = jnp.mean(xe, axis=-1, keepdims=True)
    cen = xe - mu
    var = jnp.mean(cen * cen, axis=-1, keepdims=True)
    hh = cen * jax.lax.rsqrt(var + 1e-5) * ls_ref[0, 0] + lb_ref[0, 0]
    q = D // 8
    w1s = (w1a_ref, w1b_ref, w1c_ref, w1d_ref, w1e_ref, w1f_ref, w1g_ref,
           w1h_ref)
    hh = sum(jnp.dot(hh[:, i * q:(i + 1) * q], w1s[i][0, 0],
                     preferred_element_type=_F32)
             for i in range(8)) + b1_ref[0, 0]
    hh = _gelu(hh)
    w2s = (w2a_ref, w2b_ref, w2c_ref, w2d_ref)
    hh = jnp.concatenate(
        [jnp.dot(hh, w2s[i][0, 0], preferred_element_type=_F32)
         for i in range(4)], axis=-1) + b2_ref[0, 0]
    xe = xe + hh
    xe_ref[...] = xe

    @pl.when(nb == NB - 1)
    def _acc():
        w = wdense_ref[...]
        lane = jax.lax.broadcasted_iota(jnp.int32, w.shape, 1)
        wcol = jnp.sum(jnp.where(lane == e, w, 0.0), axis=-1, keepdims=True)
        out_ref[...] += wcol * xe


def _head_kernel(c_ref, w1a_ref, w1b_ref, w1c_ref, w1d_ref, w1e_ref, w1f_ref,
                 w1g_ref, w1h_ref, b1_ref, w2a_ref, w2b_ref, w2c_ref,
                 w2d_ref, b2_ref, out_ref):
    c = c_ref[...]
    q = D // 8
    w1s = (w1a_ref, w1b_ref, w1c_ref, w1d_ref, w1e_ref, w1f_ref, w1g_ref,
           w1h_ref)
    ph = sum(jnp.dot(c[:, i * q:(i + 1) * q], w1s[i][...],
                     preferred_element_type=_F32)
             for i in range(8)) + b1_ref[...]
    ph = _gelu(ph)
    q2 = D // 4
    w2s = (w2a_ref, w2b_ref, w2c_ref, w2d_ref)
    out_ref[...] = sum(
        jnp.dot(ph[:, i * q2:(i + 1) * q2], w2s[i][...],
                preferred_element_type=_F32)
        for i in range(4)) + b2_ref[...]


@jax.jit
def kernel(ecfp_count_fp, gate_W1, gate_b1, gate_W2, gate_b2, gate_W3,
           gate_b3, expert_bias, ln_scale, ln_bias, eW1, eb1, eW2, eb2,
           pW1, pb1, pW2, pb2):
    x = ecfp_count_fp

    full = lambda shape: pl.BlockSpec(shape, lambda e, nb: (0,) * len(shape))
    per_eb = lambda shape: pl.BlockSpec(
        (1, 1) + shape, lambda e, nb: (e, nb) + (0,) * len(shape))
    g_slab = lambda i: pl.BlockSpec((D // 4, 512), lambda e, nb: (i, 0))
    w1_slab = lambda i: pl.BlockSpec(
        (1, 1, D // 8, H), lambda e, nb: (e, nb, i, 0))
    w2_slab = lambda i: pl.BlockSpec(
        (1, 1, H, D // 4), lambda e, nb: (e, nb, 0, i))

    combined, all_weights, top_i = pl.pallas_call(
        _main_kernel,
        grid=(E, NB),
        in_specs=[
            full((N, D)),
            g_slab(0), g_slab(1), g_slab(2), g_slab(3),
            full((1, 512)),
            full((512, 128)),
            full((1, 128)),
            full((128, E)),
            full((1, E)),
            full((1, E)),
            per_eb((1, D)),
            per_eb((1, D)),
            w1_slab(0), w1_slab(1), w1_slab(2), w1_slab(3),
            w1_slab(4), w1_slab(5), w1_slab(6), w1_slab(7),
            per_eb((1, H)),
            w2_slab(0), w2_slab(1), w2_slab(2), w2_slab(3),
            per_eb((1, D)),
        ],
        out_specs=(full((N, D)), full((N, E)), full((N, TOPK))),
        out_shape=(
            jax.ShapeDtypeStruct((N, D), _F32),
            jax.ShapeDtypeStruct((N, E), _F32),
            jax.ShapeDtypeStruct((N, TOPK), jnp.int32),
        ),
        scratch_shapes=[
            pltpu.VMEM((N, D), _F32),
            pltpu.VMEM((N, E), _F32),
        ],
    )(x, gate_W1, gate_W1, gate_W1, gate_W1, gate_b1.reshape(1, -1),
      gate_W2, gate_b2.reshape(1, -1), gate_W3, gate_b3.reshape(1, -1),
      expert_bias.reshape(1, -1),
      ln_scale.reshape(E, NB, 1, D), ln_bias.reshape(E, NB, 1, D),
      eW1, eW1, eW1, eW1, eW1, eW1, eW1, eW1, eb1.reshape(E, NB, 1, H),
      eW2, eW2, eW2, eW2, eb2.reshape(E, NB, 1, D))

    h_full = lambda shape: pl.BlockSpec(shape, lambda g: (0,) * len(shape))
    p1_slab = lambda i: pl.BlockSpec((D // 8, D), lambda g: (i, 0))
    p2_slab = lambda i: pl.BlockSpec((D // 4, OUT), lambda g: (i, 0))
    spectrum = pl.pallas_call(
        _head_kernel,
        grid=(1,),
        in_specs=[
            h_full((N, D)),
            p1_slab(0), p1_slab(1), p1_slab(2), p1_slab(3),
            p1_slab(4), p1_slab(5), p1_slab(6), p1_slab(7),
            h_full((1, D)),
            p2_slab(0), p2_slab(1), p2_slab(2), p2_slab(3),
            h_full((1, OUT)),
        ],
        out_specs=h_full((N, OUT)),
        out_shape=jax.ShapeDtypeStruct((N, OUT), _F32),
    )(combined, pW1, pW1, pW1, pW1, pW1, pW1, pW1, pW1,
      pb1.reshape(1, -1), pW2, pW2, pW2, pW2, pb2.reshape(1, -1))

    return (spectrum, all_weights, top_i)
